# baseline (speedup 1.0000x reference)
import functools
import jax
import jax.numpy as jnp
from jax import lax
from jax.experimental import pallas as pl
from jax.experimental.pallas import tpu as pltpu
from jax.experimental.pallas import tpu_sc as plsc

MAX_SIZE = 1000
ROWS_PER_CHUNK = 2
NBUF = 2


def kernel(x):
    B, F = x.shape
    nc, ns = 2, 16
    nw = nc * ns
    rows_pw = B // nw
    nchunks = rows_pw // ROWS_PER_CHUNK
    pairs = ROWS_PER_CHUNK * F
    pairs_pad = ((pairs + 15) // 16) * 16
    chunk_words = ROWS_PER_CHUNK * F * MAX_SIZE
    idx_words = nchunks * pairs_pad

    xpad = jnp.pad(
        x.reshape(nw, nchunks, pairs), ((0, 0), (0, 0), (0, pairs_pad - pairs))
    ).reshape(nw * idx_words)

    mesh = plsc.VectorSubcoreMesh(
        core_axis_name="c", subcore_axis_name="s", num_cores=nc, num_subcores=ns
    )

    @functools.partial(
        pl.kernel,
        mesh=mesh,
        compiler_params=pltpu.CompilerParams(
            needs_layout_passes=False, use_tc_tiling_on_sc=False
        ),
        out_type=jax.ShapeDtypeStruct((B, F, MAX_SIZE), jnp.float32),
        scratch_types=[
            pltpu.VMEM((idx_words,), jnp.int32),
            pltpu.VMEM((ROWS_PER_CHUNK, F, MAX_SIZE), jnp.float32),
            pltpu.VMEM((ROWS_PER_CHUNK, F, MAX_SIZE), jnp.float32),
            pltpu.SemaphoreType.DMA,
            pltpu.SemaphoreType.DMA,
        ],
    )
    def onehot(xpad_hbm, zeros_hbm, out_hbm, idx_v, buf0, buf1, sem0, sem1):
        wid = lax.axis_index("s") * nc + lax.axis_index("c")
        brow0 = wid * rows_pw
        pltpu.sync_copy(xpad_hbm.at[pl.ds(wid * idx_words, idx_words)], idx_v)

        bufs = (buf0, buf1)
        sems = (sem0, sem1)
        ones16 = jnp.ones((16,), jnp.float32)
        zeros16 = jnp.zeros((16,), jnp.float32)
        iota16 = lax.iota(jnp.int32, 16)

        for b in range(NBUF):
            pltpu.sync_copy(zeros_hbm, bufs[b])

        def scatter(cc, buf, vals):
            for s in range(pairs_pad // 16):
                j = iota16 + (s * 16)
                row = j // F
                feat = j - row * F
                vocab = idx_v[pl.ds(cc * pairs_pad + s * 16, 16)]
                msk = j < pairs
                plsc.store_scatter(buf, [row, feat, vocab], vals, mask=msk)

        def start_dma(cc, b):
            pltpu.async_copy(
                bufs[b],
                out_hbm.at[pl.ds(brow0 + cc * ROWS_PER_CHUNK, ROWS_PER_CHUNK)],
                sems[b],
            )

        def wait_dma(b):
            pltpu.make_async_copy(
                bufs[b], out_hbm.at[pl.ds(0, ROWS_PER_CHUNK)], sems[b]
            ).wait()

        for b in range(NBUF):
            scatter(b, bufs[b], ones16)
            start_dma(b, b)

        def body(i, _):
            for b in range(NBUF):
                cc = NBUF + i * NBUF + b
                wait_dma(b)
                scatter(cc - NBUF, bufs[b], zeros16)
                scatter(cc, bufs[b], ones16)
                start_dma(cc, b)
            return 0

        lax.fori_loop(0, (nchunks - NBUF) // NBUF, body, 0)

        for b in range(NBUF):
            wait_dma(b)

    zeros_buf = jnp.zeros((ROWS_PER_CHUNK, F, MAX_SIZE), jnp.float32)
    return onehot(xpad, zeros_buf)


# speedup vs baseline: 9.8014x; 9.8014x over previous
import jax
import jax.numpy as jnp
from jax import lax
from jax.experimental import pallas as pl

MAX_SIZE = 1000


def _onehot_block(x_ref, o_ref):
    xv = x_ref[0]
    v = lax.broadcasted_iota(jnp.int32, (26, 125, 1, 8, 128), 1) * 8 + \
        lax.broadcasted_iota(jnp.int32, (26, 125, 1, 8, 128), 3)
    o_ref[...] = (xv[:, None, None, None, :] == v).astype(jnp.float32)


def kernel(x):
    B, F = x.shape
    nbt = B // 128
    x3 = x.reshape(nbt, 128, F).transpose(0, 2, 1)
    p = pl.pallas_call(
        _onehot_block,
        grid=(nbt,),
        in_specs=[pl.BlockSpec((1, F, 128), lambda i: (i, 0, 0))],
        out_specs=pl.BlockSpec(
            (F, MAX_SIZE // 8, 1, 8, 128), lambda i: (0, 0, i, 0, 0)
        ),
        out_shape=jax.ShapeDtypeStruct((F, MAX_SIZE // 8, nbt, 8, 128), jnp.float32),
    )(x3)
    return p.transpose(2, 4, 0, 1, 3).reshape(B, F, MAX_SIZE)
